# baseline (speedup 1.0000x reference)
import functools

import jax
import jax.numpy as jnp
from jax import lax
from jax.experimental import pallas as pl
from jax.experimental.pallas import tpu as pltpu
from jax.experimental.pallas import tpu_sc as plsc

_L = 16
_W = 128
_CHUNK = 128
_NBUF = 2


def _scores_sc(user_ids, item_ids, user_pairs, item_pairs):
    B = user_ids.shape[0]
    D = _W // 2
    info = plsc.get_sparse_core_info()
    nw = info.num_cores * info.num_subcores
    b_per_w = B // nw
    n_chunks = b_per_w // _CHUNK

    mesh = plsc.VectorSubcoreMesh(core_axis_name="c", subcore_axis_name="s")

    @functools.partial(
        pl.kernel,
        mesh=mesh,
        compiler_params=pltpu.CompilerParams(needs_layout_passes=False),
        out_type=jax.ShapeDtypeStruct((B,), jnp.float32),
        scratch_types=[
            pltpu.VMEM((b_per_w,), jnp.int32),
            pltpu.VMEM((b_per_w,), jnp.int32),
            pltpu.VMEM((b_per_w,), jnp.int32),
            pltpu.VMEM((b_per_w,), jnp.int32),
            pltpu.VMEM((_NBUF, _CHUNK, _W), jnp.float32),
            pltpu.VMEM((_NBUF, _CHUNK, _W), jnp.float32),
            pltpu.VMEM((b_per_w,), jnp.float32),
            pltpu.SemaphoreType.DMA((_NBUF,)),
            pltpu.SemaphoreType.DMA((_NBUF,)),
        ],
    )
    def k(uids_hbm, iids_hbm, utab_hbm, itab_hbm, out_hbm,
          uid_v, iid_v, upair_v, ipair_v, ubuf, ibuf, out_v, sem_u, sem_i):
        wid = lax.axis_index("s") * info.num_cores + lax.axis_index("c")
        base = wid * b_per_w
        pltpu.sync_copy(uids_hbm.at[pl.ds(base, b_per_w)], uid_v)
        pltpu.sync_copy(iids_hbm.at[pl.ds(base, b_per_w)], iid_v)

        def idx_body(j, carry):
            s = pl.ds(j * _L, _L)
            upair_v[s] = uid_v[s] >> 1
            ipair_v[s] = iid_v[s] >> 1
            return carry

        lax.fori_loop(0, b_per_w // _L, idx_body, 0)

        def start(g, buf):
            s = pl.ds(g * _CHUNK, _CHUNK)
            cu = pltpu.async_copy(utab_hbm.at[upair_v.at[s]], ubuf.at[buf],
                                  sem_u.at[buf])
            ci = pltpu.async_copy(itab_hbm.at[ipair_v.at[s]], ibuf.at[buf],
                                  sem_i.at[buf])
            return cu, ci

        def drain(g, buf):
            pltpu.make_async_copy(utab_hbm.at[upair_v.at[pl.ds(0, _CHUNK)]],
                                  ubuf.at[buf], sem_u.at[buf]).wait()
            pltpu.make_async_copy(itab_hbm.at[ipair_v.at[pl.ds(0, _CHUNK)]],
                                  ibuf.at[buf], sem_i.at[buf]).wait()

        lane = lax.iota(jnp.int32, _L)

        def compute(g, buf):
            out0 = g * _CHUNK

            def blk_body(blk, carry):
                r0 = out0 + blk * _L
                rows = blk * _L + lane
                uphase = ((uid_v[pl.ds(r0, _L)] & 1) * D).astype(jnp.int32)
                iphase = ((iid_v[pl.ds(r0, _L)] & 1) * D).astype(jnp.int32)
                acc = jnp.zeros((_L,), jnp.float32)
                for d in range(D):
                    u = plsc.load_gather(ubuf.at[buf], [rows, uphase + d])
                    v = plsc.load_gather(ibuf.at[buf], [rows, iphase + d])
                    acc = acc + u * v
                out_v[pl.ds(r0, _L)] = acc
                return carry

            lax.fori_loop(0, _CHUNK // _L, blk_body, 0)

        start(0, 0)
        for g in range(n_chunks):
            if g + 1 < n_chunks:
                start(g + 1, (g + 1) % _NBUF)
            drain(g, g % _NBUF)
            compute(g, g % _NBUF)

        pltpu.sync_copy(out_v, out_hbm.at[pl.ds(base, b_per_w)])

    return k(user_ids, item_ids, user_pairs, item_pairs)


def kernel(user_ids, item_ids, user_table, item_table):
    B = user_ids.shape[0]
    user_pairs = user_table.reshape(-1, _W)
    item_pairs = item_table.reshape(-1, _W)
    scores = _scores_sc(user_ids.astype(jnp.int32), item_ids.astype(jnp.int32),
                        user_pairs, item_pairs)
    return scores.reshape(B, 1)
